# baseline (speedup 1.0000x reference)
import jax
import jax.numpy as jnp
from jax import lax
from jax.experimental import pallas as pl
from jax.experimental.pallas import tpu as pltpu
from jax.experimental.pallas import tpu_sc as plsc

N = 10000
E = 320000
D = 128
DH = D // 2
NC, NS = 2, 16
CH = 128
CPW = 160
NB = 2
EPAD = NS * CPW * CH
NPAD = 10112
RPT = NPAD // NS


def _sc_body(x0_hbm, x1_hbm, src_hbm, dst_hbm, zacc_hbm, zdeg_hbm, ones_hbm,
             acc_out, deg_out,
             src_v, dst_v, ones_v, rows0, rows1,
             acc_sh, deg_sh,
             g0, g1, s0, s1):
    c = lax.axis_index("c")
    s = lax.axis_index("s")
    rows = (rows0, rows1)
    gsem = (g0, g1)
    ssem = (s0, s1)

    def fire_gather(i, j):
        @pl.when(c == 0)
        def _():
            pltpu.async_copy(x0_hbm.at[src_v.at[i]], rows[j], gsem[j])

        @pl.when(c == 1)
        def _():
            pltpu.async_copy(x1_hbm.at[src_v.at[i]], rows[j], gsem[j])

    def wait_gather(i, j):
        pltpu.make_async_copy(x0_hbm.at[src_v.at[i]], rows[j], gsem[j]).wait()

    def fire_scatter(i, j):
        pltpu.async_copy(rows[j], acc_sh.at[dst_v.at[i]], ssem[j], add=True)

    def wait_scatter(i, j):
        pltpu.make_async_copy(rows[j], acc_sh.at[dst_v.at[i]], ssem[j]).wait()

    def handle(i, j):
        wait_gather(i, j)
        fire_scatter(i, j)
        @pl.when(c == j % 2)
        def _():
            pltpu.sync_copy(ones_v, deg_sh.at[dst_v.at[i]], add=True)

    pltpu.sync_copy(src_hbm.at[s], src_v)
    pltpu.sync_copy(dst_hbm.at[s], dst_v)
    pltpu.sync_copy(ones_hbm, ones_v)
    for j in range(NB):
        fire_gather(j, j)
    pltpu.sync_copy(zacc_hbm.at[pl.ds(s * RPT, RPT)],
                    acc_sh.at[pl.ds(s * RPT, RPT)])
    pltpu.sync_copy(zdeg_hbm.at[pl.ds(s * RPT, RPT)],
                    deg_sh.at[pl.ds(s * RPT, RPT)])
    plsc.subcore_barrier()

    def body(g, carry):
        i = g * NB
        for j in range(NB):
            handle(i + j, j)
        for j in range(NB):
            wait_scatter(i + j, j)
            fire_gather(i + j + NB, j)
        return carry

    lax.fori_loop(0, CPW // NB - 1, body, 0)
    i = CPW - NB
    for j in range(NB):
        handle(i + j, j)
    for j in range(NB):
        wait_scatter(i + j, j)
    plsc.subcore_barrier()
    pltpu.sync_copy(acc_sh.at[pl.ds(s * RPT, RPT)],
                    acc_out.at[c, pl.ds(s * RPT, RPT)])
    pltpu.sync_copy(deg_sh.at[pl.ds(s * RPT, RPT)],
                    deg_out.at[c, pl.ds(s * RPT, RPT)])


def _sc_aggregate(x0, x1, src3d, dst3d, zacc, zdeg, ones):
    mesh = plsc.VectorSubcoreMesh(core_axis_name="c", subcore_axis_name="s")
    out_type = (jax.ShapeDtypeStruct((NC, NPAD, DH), jnp.float32),
                jax.ShapeDtypeStruct((NC, NPAD, 16), jnp.float32))
    kern = pl.kernel(
        _sc_body,
        out_type=out_type,
        mesh=mesh,
        compiler_params=pltpu.CompilerParams(use_tc_tiling_on_sc=False),
        scratch_types=(
            [pltpu.VMEM((CPW, CH), jnp.int32)] * 2
            + [pltpu.VMEM((CH, 16), jnp.float32)]
            + [pltpu.VMEM((CH, DH), jnp.float32)] * NB
            + [pltpu.VMEM_SHARED((NPAD, DH), jnp.float32),
               pltpu.VMEM_SHARED((NPAD, 16), jnp.float32)]
            + [pltpu.SemaphoreType.DMA] * (2 * NB)
        ),
    )
    return kern(x0, x1, src3d, dst3d, zacc, zdeg, ones)


def _tc_body(x_ref, acc_ref, deg_ref, ws_ref, wn_ref, b_ref, o_ref):
    deg = deg_ref[0, :, 0:1] + deg_ref[1, :, 0:1]
    inv = 1.0 / jnp.maximum(deg, 1.0)
    mean = jnp.concatenate([acc_ref[0], acc_ref[1]], axis=1) * inv
    o_ref[...] = (
        jnp.dot(x_ref[...], ws_ref[...],
                preferred_element_type=jnp.float32,
                precision=lax.Precision.HIGHEST)
        + jnp.dot(mean, wn_ref[...],
                  preferred_element_type=jnp.float32,
                  precision=lax.Precision.HIGHEST)
        + b_ref[...])


def _tc_combine(x, acc, deg, W_self, W_neigh, b2d):
    blk = 1000
    grid = (N // blk,)
    return pl.pallas_call(
        _tc_body,
        grid=grid,
        in_specs=[
            pl.BlockSpec((blk, D), lambda i: (i, 0)),
            pl.BlockSpec((NC, blk, DH), lambda i: (0, i, 0)),
            pl.BlockSpec((NC, blk, 16), lambda i: (0, i, 0)),
            pl.BlockSpec((D, D), lambda i: (0, 0)),
            pl.BlockSpec((D, D), lambda i: (0, 0)),
            pl.BlockSpec((1, D), lambda i: (0, 0)),
        ],
        out_specs=pl.BlockSpec((blk, D), lambda i: (i, 0)),
        out_shape=jax.ShapeDtypeStruct((N, D), jnp.float32),
    )(x, acc, deg, W_self, W_neigh, b2d)


def kernel(x, edge_index, W_self, W_neigh, b):
    src = edge_index[0].astype(jnp.int32)
    dst = edge_index[1].astype(jnp.int32)
    pad = EPAD - E
    src = jnp.concatenate([src, jnp.zeros((pad,), jnp.int32)])
    dst = jnp.concatenate([dst, jnp.full((pad,), N, jnp.int32)])
    src3d = src.reshape(NS, CPW, CH)
    dst3d = dst.reshape(NS, CPW, CH)
    x0 = x[:, :DH]
    x1 = x[:, DH:]
    zacc = jnp.zeros((NPAD, DH), jnp.float32)
    zdeg = jnp.zeros((NPAD, 16), jnp.float32)
    ones = jnp.ones((CH, 16), jnp.float32)
    acc, deg = _sc_aggregate(x0, x1, src3d, dst3d, zacc, zdeg, ones)
    return _tc_combine(x, acc, deg, W_self, W_neigh, b.reshape(1, D))


# speedup vs baseline: 1.1713x; 1.1713x over previous
import jax
import jax.numpy as jnp
from jax import lax
from jax.experimental import pallas as pl
from jax.experimental.pallas import tpu as pltpu
from jax.experimental.pallas import tpu_sc as plsc

N = 10000
E = 320000
D = 128
DH = D // 2
DC = DH + 16
NC, NS = 2, 16
CH = 128
CPW = 157
EPAD = NS * CPW * CH
NPAD = 10112
RPT = NPAD // NS


def _sc_body(x0_hbm, x1_hbm, src_hbm, dst_hbm, zacc_hbm,
             acc_out,
             src_v, dst_v, rows_v, acc_sh):
    c = lax.axis_index("c")
    s = lax.axis_index("s")
    pltpu.sync_copy(zacc_hbm.at[pl.ds(s * RPT, RPT)],
                    acc_sh.at[pl.ds(s * RPT, RPT)])
    pltpu.sync_copy(src_hbm.at[s], src_v)
    pltpu.sync_copy(dst_hbm.at[s], dst_v)
    plsc.subcore_barrier()

    def body(i, carry):
        @pl.when(c == 0)
        def _():
            pltpu.sync_copy(x0_hbm.at[src_v.at[i]], rows_v)

        @pl.when(c == 1)
        def _():
            pltpu.sync_copy(x1_hbm.at[src_v.at[i]], rows_v)

        pltpu.sync_copy(rows_v, acc_sh.at[dst_v.at[i]], add=True)
        return carry

    lax.fori_loop(0, CPW, body, 0)
    plsc.subcore_barrier()
    pltpu.sync_copy(acc_sh.at[pl.ds(s * RPT, RPT)],
                    acc_out.at[c, pl.ds(s * RPT, RPT)])


def _sc_aggregate(x0, x1, src3d, dst3d, zacc):
    mesh = plsc.VectorSubcoreMesh(core_axis_name="c", subcore_axis_name="s")
    out_type = jax.ShapeDtypeStruct((NC, NPAD, DC), jnp.float32)
    kern = pl.kernel(
        _sc_body,
        out_type=out_type,
        mesh=mesh,
        compiler_params=pltpu.CompilerParams(use_tc_tiling_on_sc=False),
        scratch_types=[
            pltpu.VMEM((CPW, CH), jnp.int32),
            pltpu.VMEM((CPW, CH), jnp.int32),
            pltpu.VMEM((CH, DC), jnp.float32),
            pltpu.VMEM_SHARED((NPAD, DC), jnp.float32),
        ],
    )
    return kern(x0, x1, src3d, dst3d, zacc)


def _tc_body(x_ref, acc_ref, ws_ref, wn_ref, b_ref, o_ref):
    deg = (acc_ref[0, :, DH:DH + 1] + acc_ref[1, :, DH:DH + 1]) * 0.5
    inv = 1.0 / jnp.maximum(deg, 1.0)
    mean = jnp.concatenate([acc_ref[0, :, :DH], acc_ref[1, :, :DH]],
                           axis=1) * inv
    o_ref[...] = (
        jnp.dot(x_ref[...], ws_ref[...],
                preferred_element_type=jnp.float32,
                precision=lax.Precision.HIGHEST)
        + jnp.dot(mean, wn_ref[...],
                  preferred_element_type=jnp.float32,
                  precision=lax.Precision.HIGHEST)
        + b_ref[...])


def _tc_combine(x, acc, W_self, W_neigh, b2d):
    blk = 1000
    grid = (N // blk,)
    return pl.pallas_call(
        _tc_body,
        grid=grid,
        in_specs=[
            pl.BlockSpec((blk, D), lambda i: (i, 0)),
            pl.BlockSpec((NC, blk, DC), lambda i: (0, i, 0)),
            pl.BlockSpec((D, D), lambda i: (0, 0)),
            pl.BlockSpec((D, D), lambda i: (0, 0)),
            pl.BlockSpec((1, D), lambda i: (0, 0)),
        ],
        out_specs=pl.BlockSpec((blk, D), lambda i: (i, 0)),
        out_shape=jax.ShapeDtypeStruct((N, D), jnp.float32),
    )(x, acc, W_self, W_neigh, b2d)


def kernel(x, edge_index, W_self, W_neigh, b):
    ei = edge_index.astype(jnp.int32)
    pad = EPAD - E
    ei = jnp.concatenate(
        [ei, jnp.concatenate([jnp.zeros((1, pad), jnp.int32),
                              jnp.full((1, pad), N, jnp.int32)])], axis=1)
    src3d = ei[0].reshape(NS, CPW, CH)
    dst3d = ei[1].reshape(NS, CPW, CH)
    ones = jnp.ones((N, 16), jnp.float32)
    x0 = jnp.concatenate([x[:, :DH], ones], axis=1)
    x1 = jnp.concatenate([x[:, DH:], ones], axis=1)
    zacc = jnp.zeros((NPAD, DC), jnp.float32)
    acc = _sc_aggregate(x0, x1, src3d, dst3d, zacc)
    return _tc_combine(x, acc, W_self, W_neigh, b.reshape(1, D))


# speedup vs baseline: 1.3172x; 1.1245x over previous
import jax
import jax.numpy as jnp
from jax import lax
from jax.experimental import pallas as pl
from jax.experimental.pallas import tpu as pltpu
from jax.experimental.pallas import tpu_sc as plsc

N = 10000
E = 320000
D = 128
DH = D // 2
NC, NS = 2, 16
CH = 128
CPW = 157
EPAD = NS * CPW * CH
NPAD = 10112
RPT = NPAD // NS


def _sc_body(x0_hbm, x1_hbm, src_hbm, dst_hbm, zacc_hbm, zdeg_hbm, ones_hbm,
             acc_out, deg_out,
             src_v, dst_v, rows_v, ones_v, acc_sh, deg_sh):
    c = lax.axis_index("c")
    s = lax.axis_index("s")
    pltpu.sync_copy(zacc_hbm, acc_sh.at[pl.ds(s * RPT, RPT)])
    pltpu.sync_copy(zdeg_hbm, deg_sh.at[pl.ds(s * RPT, RPT)])
    pltpu.sync_copy(src_hbm.at[s], src_v)
    pltpu.sync_copy(dst_hbm.at[s], dst_v)
    pltpu.sync_copy(ones_hbm, ones_v)
    plsc.subcore_barrier()

    def body(i, carry):
        @pl.when(c == 0)
        def _():
            pltpu.sync_copy(x0_hbm.at[src_v.at[i]], rows_v)

        @pl.when(c == 1)
        def _():
            pltpu.sync_copy(x1_hbm.at[src_v.at[i]], rows_v)

        pltpu.sync_copy(rows_v, acc_sh.at[dst_v.at[i]], add=True)

        @pl.when(lax.rem(i, 2) == c)
        def _():
            pltpu.sync_copy(ones_v, deg_sh.at[dst_v.at[i]], add=True)

        return carry

    lax.fori_loop(0, CPW, body, 0)
    plsc.subcore_barrier()
    pltpu.sync_copy(acc_sh.at[pl.ds(s * RPT, RPT)],
                    acc_out.at[c, pl.ds(s * RPT, RPT)])
    pltpu.sync_copy(deg_sh.at[pl.ds(s * RPT, RPT)],
                    deg_out.at[c, pl.ds(s * RPT, RPT)])


def _sc_aggregate(x0, x1, src3d, dst3d, zacc, zdeg, ones):
    mesh = plsc.VectorSubcoreMesh(core_axis_name="c", subcore_axis_name="s")
    out_type = (jax.ShapeDtypeStruct((NC, NPAD, DH), jnp.float32),
                jax.ShapeDtypeStruct((NC, NPAD, 16), jnp.float32))
    kern = pl.kernel(
        _sc_body,
        out_type=out_type,
        mesh=mesh,
        compiler_params=pltpu.CompilerParams(use_tc_tiling_on_sc=False),
        scratch_types=[
            pltpu.VMEM((CPW, CH), jnp.int32),
            pltpu.VMEM((CPW, CH), jnp.int32),
            pltpu.VMEM((CH, DH), jnp.float32),
            pltpu.VMEM((CH, 16), jnp.float32),
            pltpu.VMEM_SHARED((NPAD, DH), jnp.float32),
            pltpu.VMEM_SHARED((NPAD, 16), jnp.float32),
        ],
    )
    return kern(x0, x1, src3d, dst3d, zacc, zdeg, ones)


def _tc_body(x_ref, acc_ref, deg_ref, ws_ref, wn_ref, b_ref, o_ref):
    deg = deg_ref[0, :, 0:1] + deg_ref[1, :, 0:1]
    inv = 1.0 / jnp.maximum(deg, 1.0)
    mean = jnp.concatenate([acc_ref[0], acc_ref[1]], axis=1) * inv
    o_ref[...] = (
        jnp.dot(x_ref[...], ws_ref[...],
                preferred_element_type=jnp.float32,
                precision=lax.Precision.HIGHEST)
        + jnp.dot(mean, wn_ref[...],
                  preferred_element_type=jnp.float32,
                  precision=lax.Precision.HIGHEST)
        + b_ref[...])


def _tc_combine(x, acc, deg, W_self, W_neigh, b2d):
    blk = 1000
    grid = (N // blk,)
    return pl.pallas_call(
        _tc_body,
        grid=grid,
        in_specs=[
            pl.BlockSpec((blk, D), lambda i: (i, 0)),
            pl.BlockSpec((NC, blk, DH), lambda i: (0, i, 0)),
            pl.BlockSpec((NC, blk, 16), lambda i: (0, i, 0)),
            pl.BlockSpec((D, D), lambda i: (0, 0)),
            pl.BlockSpec((D, D), lambda i: (0, 0)),
            pl.BlockSpec((1, D), lambda i: (0, 0)),
        ],
        out_specs=pl.BlockSpec((blk, D), lambda i: (i, 0)),
        out_shape=jax.ShapeDtypeStruct((N, D), jnp.float32),
    )(x, acc, deg, W_self, W_neigh, b2d)


def kernel(x, edge_index, W_self, W_neigh, b):
    ei = edge_index.astype(jnp.int32)
    pad = EPAD - E
    ei = jnp.concatenate(
        [ei, jnp.concatenate([jnp.zeros((1, pad), jnp.int32),
                              jnp.full((1, pad), N, jnp.int32)])], axis=1)
    src3d = ei[0].reshape(NS, CPW, CH)
    dst3d = ei[1].reshape(NS, CPW, CH)
    x0 = x[:, :DH]
    x1 = x[:, DH:]
    zacc = jnp.zeros((RPT, DH), jnp.float32)
    zdeg = jnp.zeros((RPT, 16), jnp.float32)
    ones = jnp.ones((CH, 16), jnp.float32)
    acc, deg = _sc_aggregate(x0, x1, src3d, dst3d, zacc, zdeg, ones)
    return _tc_combine(x, acc, deg, W_self, W_neigh, b.reshape(1, D))


# speedup vs baseline: 1.3243x; 1.0054x over previous
import jax
import jax.numpy as jnp
from jax import lax
from jax.experimental import pallas as pl
from jax.experimental.pallas import tpu as pltpu
from jax.experimental.pallas import tpu_sc as plsc

N = 10000
E = 320000
D = 128
DH = D // 2
NC, NS = 2, 16
CH = 128
CPW = 157
EPAD = NS * CPW * CH
NPAD = 10112
RPT = NPAD // NS


def _sc_body(x0_hbm, x1_hbm, src_hbm, dst_hbm, zacc_hbm, zdeg_hbm, ones_hbm,
             acc_out, deg_out,
             src_v, dst_v, rows_v, ones_v, acc_sh, deg_sh):
    c = lax.axis_index("c")
    s = lax.axis_index("s")
    pltpu.sync_copy(zacc_hbm, acc_sh.at[pl.ds(s * RPT, RPT)])
    pltpu.sync_copy(zdeg_hbm, deg_sh.at[pl.ds(s * RPT, RPT)])
    pltpu.sync_copy(src_hbm.at[s], src_v)
    pltpu.sync_copy(dst_hbm.at[s], dst_v)
    pltpu.sync_copy(ones_hbm, ones_v)
    plsc.subcore_barrier()

    def body(i, carry):
        @pl.when(c == 0)
        def _():
            pltpu.sync_copy(x0_hbm.at[src_v.at[i]], rows_v)

        @pl.when(c == 1)
        def _():
            pltpu.sync_copy(x1_hbm.at[src_v.at[i]], rows_v)

        pltpu.sync_copy(rows_v, acc_sh.at[dst_v.at[i]], add=True)

        @pl.when(lax.rem(i, 2) == c)
        def _():
            pltpu.sync_copy(ones_v, deg_sh.at[dst_v.at[i]], add=True)

        return carry

    lax.fori_loop(0, CPW, body, 0)
    plsc.subcore_barrier()
    pltpu.sync_copy(acc_sh.at[pl.ds(s * RPT, RPT)],
                    acc_out.at[c, pl.ds(s * RPT, RPT)])
    pltpu.sync_copy(deg_sh.at[pl.ds(s * RPT, RPT)],
                    deg_out.at[c, pl.ds(s * RPT, RPT)])


def _sc_aggregate(x0, x1, src3d, dst3d, zacc, zdeg, ones):
    mesh = plsc.VectorSubcoreMesh(core_axis_name="c", subcore_axis_name="s")
    out_type = (jax.ShapeDtypeStruct((NC, NPAD, DH), jnp.float32),
                jax.ShapeDtypeStruct((NC, NPAD, 16), jnp.float32))
    kern = pl.kernel(
        _sc_body,
        out_type=out_type,
        mesh=mesh,
        compiler_params=pltpu.CompilerParams(use_tc_tiling_on_sc=False),
        scratch_types=[
            pltpu.VMEM((CPW, CH), jnp.int32),
            pltpu.VMEM((CPW, CH), jnp.int32),
            pltpu.VMEM((CH, DH), jnp.float32),
            pltpu.VMEM((CH, 16), jnp.float32),
            pltpu.VMEM_SHARED((NPAD, DH), jnp.float32),
            pltpu.VMEM_SHARED((NPAD, 16), jnp.float32),
        ],
    )
    return kern(x0, x1, src3d, dst3d, zacc, zdeg, ones)


def _tc_self_body(x_ref, ws_ref, b_ref, o_ref):
    o_ref[...] = jnp.dot(x_ref[...], ws_ref[...],
                         preferred_element_type=jnp.float32,
                         precision=lax.Precision.HIGHEST) + b_ref[...]


def _tc_self(x, W_self, b2d):
    blk = 1000
    return pl.pallas_call(
        _tc_self_body,
        grid=(N // blk,),
        in_specs=[
            pl.BlockSpec((blk, D), lambda i: (i, 0)),
            pl.BlockSpec((D, D), lambda i: (0, 0)),
            pl.BlockSpec((1, D), lambda i: (0, 0)),
        ],
        out_specs=pl.BlockSpec((blk, D), lambda i: (i, 0)),
        out_shape=jax.ShapeDtypeStruct((N, D), jnp.float32),
    )(x, W_self, b2d)


def _tc_body(y_ref, acc_ref, deg_ref, wn_ref, o_ref):
    deg = deg_ref[0, :, 0:1] + deg_ref[1, :, 0:1]
    inv = 1.0 / jnp.maximum(deg, 1.0)
    mean = jnp.concatenate([acc_ref[0], acc_ref[1]], axis=1) * inv
    o_ref[...] = y_ref[...] + jnp.dot(mean, wn_ref[...],
                                      preferred_element_type=jnp.float32,
                                      precision=lax.Precision.HIGHEST)


def _tc_combine(y, acc, deg, W_neigh):
    blk = 1000
    grid = (N // blk,)
    return pl.pallas_call(
        _tc_body,
        grid=grid,
        in_specs=[
            pl.BlockSpec((blk, D), lambda i: (i, 0)),
            pl.BlockSpec((NC, blk, DH), lambda i: (0, i, 0)),
            pl.BlockSpec((NC, blk, 16), lambda i: (0, i, 0)),
            pl.BlockSpec((D, D), lambda i: (0, 0)),
        ],
        out_specs=pl.BlockSpec((blk, D), lambda i: (i, 0)),
        out_shape=jax.ShapeDtypeStruct((N, D), jnp.float32),
    )(y, acc, deg, W_neigh)


def kernel(x, edge_index, W_self, W_neigh, b):
    ei = edge_index.astype(jnp.int32)
    pad = EPAD - E
    ei = jnp.concatenate(
        [ei, jnp.concatenate([jnp.zeros((1, pad), jnp.int32),
                              jnp.full((1, pad), N, jnp.int32)])], axis=1)
    src3d = ei[0].reshape(NS, CPW, CH)
    dst3d = ei[1].reshape(NS, CPW, CH)
    x0 = x[:, :DH]
    x1 = x[:, DH:]
    zacc = jnp.zeros((RPT, DH), jnp.float32)
    zdeg = jnp.zeros((RPT, 16), jnp.float32)
    ones = jnp.ones((CH, 16), jnp.float32)
    y = _tc_self(x, W_self, b.reshape(1, D))
    acc, deg = _sc_aggregate(x0, x1, src3d, dst3d, zacc, zdeg, ones)
    return _tc_combine(y, acc, deg, W_neigh)


# speedup vs baseline: 1.5571x; 1.1758x over previous
import jax
import jax.numpy as jnp
from jax import lax
from jax.experimental import pallas as pl
from jax.experimental.pallas import tpu as pltpu
from jax.experimental.pallas import tpu_sc as plsc

N = 10000
E = 320000
D = 128
DH = D // 2
NC, NS = 2, 16
CH = 128
CPW = 158
EPAD = NS * CPW * CH
NPAD = 10112
RPT = NPAD // NS


def _sc_body(x0_hbm, x1_hbm, src_hbm, dst_hbm, zacc_hbm, zdeg_hbm, ones_hbm,
             acc_out, deg_out,
             src_v, dst_v, rows_a, rows_b, ones_v, acc_sh, deg_sh,
             gsem_a, gsem_b):
    c = lax.axis_index("c")
    s = lax.axis_index("s")
    pltpu.sync_copy(zacc_hbm, acc_sh.at[pl.ds(s * RPT, RPT)])
    pltpu.sync_copy(zdeg_hbm, deg_sh.at[pl.ds(s * RPT, RPT)])
    pltpu.sync_copy(src_hbm.at[s], src_v)
    pltpu.sync_copy(dst_hbm.at[s], dst_v)
    pltpu.sync_copy(ones_hbm, ones_v)

    def fire(i, rows, sem):
        @pl.when(c == 0)
        def _():
            pltpu.async_copy(x0_hbm.at[src_v.at[i]], rows, sem)

        @pl.when(c == 1)
        def _():
            pltpu.async_copy(x1_hbm.at[src_v.at[i]], rows, sem)

    def drain(i, rows, sem, par):
        pltpu.make_async_copy(x0_hbm.at[src_v.at[i]], rows, sem).wait()
        pltpu.sync_copy(rows, acc_sh.at[dst_v.at[i]], add=True)

        @pl.when(par == c)
        def _():
            pltpu.sync_copy(ones_v, deg_sh.at[dst_v.at[i]], add=True)

    fire(0, rows_a, gsem_a)
    plsc.subcore_barrier()

    def body(g, carry):
        i = g * 2
        fire(i + 1, rows_b, gsem_b)
        drain(i, rows_a, gsem_a, 0)

        @pl.when(i + 2 < CPW)
        def _():
            fire(i + 2, rows_a, gsem_a)

        drain(i + 1, rows_b, gsem_b, 1)
        return carry

    lax.fori_loop(0, CPW // 2, body, 0)
    plsc.subcore_barrier()
    pltpu.sync_copy(acc_sh.at[pl.ds(s * RPT, RPT)],
                    acc_out.at[c, pl.ds(s * RPT, RPT)])
    pltpu.sync_copy(deg_sh.at[pl.ds(s * RPT, RPT)],
                    deg_out.at[c, pl.ds(s * RPT, RPT)])


def _sc_aggregate(x0, x1, src3d, dst3d, zacc, zdeg, ones):
    mesh = plsc.VectorSubcoreMesh(core_axis_name="c", subcore_axis_name="s")
    out_type = (jax.ShapeDtypeStruct((NC, NPAD, DH), jnp.float32),
                jax.ShapeDtypeStruct((NC, NPAD, 16), jnp.float32))
    kern = pl.kernel(
        _sc_body,
        out_type=out_type,
        mesh=mesh,
        compiler_params=pltpu.CompilerParams(use_tc_tiling_on_sc=False),
        scratch_types=[
            pltpu.VMEM((CPW, CH), jnp.int32),
            pltpu.VMEM((CPW, CH), jnp.int32),
            pltpu.VMEM((CH, DH), jnp.float32),
            pltpu.VMEM((CH, DH), jnp.float32),
            pltpu.VMEM((CH, 16), jnp.float32),
            pltpu.VMEM_SHARED((NPAD, DH), jnp.float32),
            pltpu.VMEM_SHARED((NPAD, 16), jnp.float32),
            pltpu.SemaphoreType.DMA,
            pltpu.SemaphoreType.DMA,
        ],
    )
    return kern(x0, x1, src3d, dst3d, zacc, zdeg, ones)


def _tc_self_body(x_ref, ws_ref, b_ref, o_ref):
    o_ref[...] = jnp.dot(x_ref[...], ws_ref[...],
                         preferred_element_type=jnp.float32,
                         precision=lax.Precision.HIGHEST) + b_ref[...]


def _tc_self(x, W_self, b2d):
    blk = 1000
    return pl.pallas_call(
        _tc_self_body,
        grid=(N // blk,),
        in_specs=[
            pl.BlockSpec((blk, D), lambda i: (i, 0)),
            pl.BlockSpec((D, D), lambda i: (0, 0)),
            pl.BlockSpec((1, D), lambda i: (0, 0)),
        ],
        out_specs=pl.BlockSpec((blk, D), lambda i: (i, 0)),
        out_shape=jax.ShapeDtypeStruct((N, D), jnp.float32),
    )(x, W_self, b2d)


def _tc_body(y_ref, acc_ref, deg_ref, wn_ref, o_ref):
    deg = deg_ref[0, :, 0:1] + deg_ref[1, :, 0:1]
    inv = 1.0 / jnp.maximum(deg, 1.0)
    mean = jnp.concatenate([acc_ref[0], acc_ref[1]], axis=1) * inv
    o_ref[...] = y_ref[...] + jnp.dot(mean, wn_ref[...],
                                      preferred_element_type=jnp.float32,
                                      precision=lax.Precision.HIGHEST)


def _tc_combine(y, acc, deg, W_neigh):
    blk = 1000
    grid = (N // blk,)
    return pl.pallas_call(
        _tc_body,
        grid=grid,
        in_specs=[
            pl.BlockSpec((blk, D), lambda i: (i, 0)),
            pl.BlockSpec((NC, blk, DH), lambda i: (0, i, 0)),
            pl.BlockSpec((NC, blk, 16), lambda i: (0, i, 0)),
            pl.BlockSpec((D, D), lambda i: (0, 0)),
        ],
        out_specs=pl.BlockSpec((blk, D), lambda i: (i, 0)),
        out_shape=jax.ShapeDtypeStruct((N, D), jnp.float32),
    )(y, acc, deg, W_neigh)


def kernel(x, edge_index, W_self, W_neigh, b):
    ei = edge_index.astype(jnp.int32)
    pad = EPAD - E
    ei = jnp.concatenate(
        [ei, jnp.concatenate([jnp.zeros((1, pad), jnp.int32),
                              jnp.full((1, pad), N, jnp.int32)])], axis=1)
    src3d = ei[0].reshape(NS, CPW, CH)
    dst3d = ei[1].reshape(NS, CPW, CH)
    x0 = x[:, :DH]
    x1 = x[:, DH:]
    zacc = jnp.zeros((RPT, DH), jnp.float32)
    zdeg = jnp.zeros((RPT, 16), jnp.float32)
    ones = jnp.ones((CH, 16), jnp.float32)
    y = _tc_self(x, W_self, b.reshape(1, D))
    acc, deg = _sc_aggregate(x0, x1, src3d, dst3d, zacc, zdeg, ones)
    return _tc_combine(y, acc, deg, W_neigh)


# speedup vs baseline: 1.7980x; 1.1547x over previous
import jax
import jax.numpy as jnp
from jax import lax
from jax.experimental import pallas as pl
from jax.experimental.pallas import tpu as pltpu
from jax.experimental.pallas import tpu_sc as plsc

N = 10000
E = 320000
D = 128
DH = D // 2
NC, NS = 2, 16
CH = 128
CPW = 158
EPAD = NS * CPW * CH
NPAD = 10112
RPT = NPAD // NS


def _sc_body(x0_hbm, x1_hbm, src_hbm, dst_hbm, zacc_hbm, zdeg_hbm, ones_hbm,
             acc_out, deg_out,
             src_v, dst_v, rows_a, rows_b, frows_v, ones_v, acc_sh, deg_sh,
             gsem_a, gsem_b):
    c = lax.axis_index("c")
    s = lax.axis_index("s")
    pltpu.sync_copy(zacc_hbm, acc_sh.at[pl.ds(s * RPT, RPT)])
    pltpu.sync_copy(zdeg_hbm, deg_sh.at[pl.ds(s * RPT, RPT)])
    pltpu.sync_copy(src_hbm.at[s], src_v)
    pltpu.sync_copy(dst_hbm.at[s], dst_v)
    pltpu.sync_copy(ones_hbm, ones_v)

    def fire(i, rows, sem):
        @pl.when(c == 0)
        def _():
            pltpu.async_copy(x0_hbm.at[src_v.at[i]], rows, sem)

        @pl.when(c == 1)
        def _():
            pltpu.async_copy(x1_hbm.at[src_v.at[i]], rows, sem)

    def drain(i, rows, sem, par):
        pltpu.make_async_copy(x0_hbm.at[src_v.at[i]], rows, sem).wait()

        @plsc.parallel_loop(0, CH, unroll=8)
        def _(r):
            v0 = plsc.bitcast(rows[r, pl.ds(0, 32)], jnp.int32)
            v1 = plsc.bitcast(rows[r, pl.ds(32, 32)], jnp.int32)
            hi = jnp.int32(-65536)
            frows_v[r, pl.ds(0, 16)] = plsc.bitcast(v0 << 16, jnp.float32)
            frows_v[r, pl.ds(16, 16)] = plsc.bitcast(v0 & hi, jnp.float32)
            frows_v[r, pl.ds(32, 16)] = plsc.bitcast(v1 << 16, jnp.float32)
            frows_v[r, pl.ds(48, 16)] = plsc.bitcast(v1 & hi, jnp.float32)

        pltpu.sync_copy(frows_v, acc_sh.at[dst_v.at[i]], add=True)

        @pl.when(par == c)
        def _():
            pltpu.sync_copy(ones_v, deg_sh.at[dst_v.at[i]], add=True)

    fire(0, rows_a, gsem_a)
    plsc.subcore_barrier()

    def body(g, carry):
        i = g * 2
        fire(i + 1, rows_b, gsem_b)
        drain(i, rows_a, gsem_a, 0)

        @pl.when(i + 2 < CPW)
        def _():
            fire(i + 2, rows_a, gsem_a)

        drain(i + 1, rows_b, gsem_b, 1)
        return carry

    lax.fori_loop(0, CPW // 2, body, 0)
    plsc.subcore_barrier()
    pltpu.sync_copy(acc_sh.at[pl.ds(s * RPT, RPT)],
                    acc_out.at[c, pl.ds(s * RPT, RPT)])
    pltpu.sync_copy(deg_sh.at[pl.ds(s * RPT, RPT)],
                    deg_out.at[c, pl.ds(s * RPT, RPT)])


def _sc_aggregate(x0, x1, src3d, dst3d, zacc, zdeg, ones):
    mesh = plsc.VectorSubcoreMesh(core_axis_name="c", subcore_axis_name="s")
    out_type = (jax.ShapeDtypeStruct((NC, NPAD, DH), jnp.float32),
                jax.ShapeDtypeStruct((NC, NPAD, 16), jnp.float32))
    kern = pl.kernel(
        _sc_body,
        out_type=out_type,
        mesh=mesh,
        compiler_params=pltpu.CompilerParams(use_tc_tiling_on_sc=False,
                                            needs_layout_passes=False),
        scratch_types=[
            pltpu.VMEM((CPW, CH), jnp.int32),
            pltpu.VMEM((CPW, CH), jnp.int32),
            pltpu.VMEM((CH, DH), jnp.bfloat16),
            pltpu.VMEM((CH, DH), jnp.bfloat16),
            pltpu.VMEM((CH, DH), jnp.float32),
            pltpu.VMEM((CH, 16), jnp.float32),
            pltpu.VMEM_SHARED((NPAD, DH), jnp.float32),
            pltpu.VMEM_SHARED((NPAD, 16), jnp.float32),
            pltpu.SemaphoreType.DMA,
            pltpu.SemaphoreType.DMA,
        ],
    )
    return kern(x0, x1, src3d, dst3d, zacc, zdeg, ones)


def _tc_self_body(x_ref, ws_ref, b_ref, o_ref):
    o_ref[...] = jnp.dot(x_ref[...], ws_ref[...],
                         preferred_element_type=jnp.float32,
                         precision=lax.Precision.HIGHEST) + b_ref[...]


def _tc_self(x, W_self, b2d):
    blk = 1000
    return pl.pallas_call(
        _tc_self_body,
        grid=(N // blk,),
        in_specs=[
            pl.BlockSpec((blk, D), lambda i: (i, 0)),
            pl.BlockSpec((D, D), lambda i: (0, 0)),
            pl.BlockSpec((1, D), lambda i: (0, 0)),
        ],
        out_specs=pl.BlockSpec((blk, D), lambda i: (i, 0)),
        out_shape=jax.ShapeDtypeStruct((N, D), jnp.float32),
    )(x, W_self, b2d)


def _tc_body(y_ref, acc_ref, deg_ref, wn_ref, o_ref):
    deg = deg_ref[0, :, 0:1] + deg_ref[1, :, 0:1]
    inv = 1.0 / jnp.maximum(deg, 1.0)
    mean = jnp.concatenate([acc_ref[0], acc_ref[1]], axis=1) * inv
    o_ref[...] = y_ref[...] + jnp.dot(mean, wn_ref[...],
                                      preferred_element_type=jnp.float32,
                                      precision=lax.Precision.HIGHEST)


def _tc_combine(y, acc, deg, W_neigh):
    blk = 1000
    grid = (N // blk,)
    return pl.pallas_call(
        _tc_body,
        grid=grid,
        in_specs=[
            pl.BlockSpec((blk, D), lambda i: (i, 0)),
            pl.BlockSpec((NC, blk, DH), lambda i: (0, i, 0)),
            pl.BlockSpec((NC, blk, 16), lambda i: (0, i, 0)),
            pl.BlockSpec((D, D), lambda i: (0, 0)),
        ],
        out_specs=pl.BlockSpec((blk, D), lambda i: (i, 0)),
        out_shape=jax.ShapeDtypeStruct((N, D), jnp.float32),
    )(y, acc, deg, W_neigh)


def kernel(x, edge_index, W_self, W_neigh, b):
    ei = edge_index.astype(jnp.int32)
    pad = EPAD - E
    ei = jnp.concatenate(
        [ei, jnp.concatenate([jnp.zeros((1, pad), jnp.int32),
                              jnp.full((1, pad), N, jnp.int32)])], axis=1)
    src3d = ei[0].reshape(NS, CPW, CH)
    dst3d = ei[1].reshape(NS, CPW, CH)
    x0 = x[:, :DH].astype(jnp.bfloat16)
    x1 = x[:, DH:].astype(jnp.bfloat16)
    half = jnp.concatenate([jnp.arange(0, 32, 2), jnp.arange(1, 32, 2),
                            jnp.arange(32, 64, 2), jnp.arange(33, 64, 2)])
    wperm = jnp.concatenate([half, DH + half])
    W_neigh = W_neigh[wperm, :]
    zacc = jnp.zeros((RPT, DH), jnp.float32)
    zdeg = jnp.zeros((RPT, 16), jnp.float32)
    ones = jnp.ones((CH, 16), jnp.float32)
    y = _tc_self(x, W_self, b.reshape(1, D))
    acc, deg = _sc_aggregate(x0, x1, src3d, dst3d, zacc, zdeg, ones)
    return _tc_combine(y, acc, deg, W_neigh)


# speedup vs baseline: 1.8363x; 1.0213x over previous
import jax
import jax.numpy as jnp
from jax import lax
from jax.experimental import pallas as pl
from jax.experimental.pallas import tpu as pltpu
from jax.experimental.pallas import tpu_sc as plsc

N = 10000
E = 320000
D = 128
DH = D // 2
NC, NS = 2, 16
CH = 128
CPW = 158
EPAD = NS * CPW * CH
NPAD = 10112
RPT = NPAD // NS


def _sc_body(x0_hbm, x1_hbm, src_hbm, dst_hbm, zacc_hbm, zdeg_hbm, ones_hbm,
             acc_out, deg_out,
             src_v, dst_v, rows_a, rows_b, frows_v, ones_v, acc_sh, deg_sh,
             gsem_a, gsem_b):
    c = lax.axis_index("c")
    s = lax.axis_index("s")
    pltpu.sync_copy(zacc_hbm, acc_sh.at[pl.ds(s * RPT, RPT)])
    pltpu.sync_copy(zdeg_hbm, deg_sh.at[pl.ds(s * RPT, RPT)])
    pltpu.sync_copy(src_hbm.at[s], src_v)
    pltpu.sync_copy(dst_hbm.at[s], dst_v)
    pltpu.sync_copy(ones_hbm, ones_v)

    def fire(i, rows, sem):
        @pl.when(c == 0)
        def _():
            pltpu.async_copy(x0_hbm.at[src_v.at[i]], rows, sem)

        @pl.when(c == 1)
        def _():
            pltpu.async_copy(x1_hbm.at[src_v.at[i]], rows, sem)

    def drain(i, rows, sem, par):
        @pl.when(par == c)
        def _():
            pltpu.sync_copy(ones_v, deg_sh.at[dst_v.at[i]], add=True)

        pltpu.make_async_copy(x0_hbm.at[src_v.at[i]], rows, sem).wait()

        @plsc.parallel_loop(0, CH, unroll=8)
        def _(r):
            v0 = plsc.bitcast(rows[r, pl.ds(0, 32)], jnp.int32)
            v1 = plsc.bitcast(rows[r, pl.ds(32, 32)], jnp.int32)
            hi = jnp.int32(-65536)
            frows_v[r, pl.ds(0, 16)] = plsc.bitcast(v0 << 16, jnp.float32)
            frows_v[r, pl.ds(16, 16)] = plsc.bitcast(v0 & hi, jnp.float32)
            frows_v[r, pl.ds(32, 16)] = plsc.bitcast(v1 << 16, jnp.float32)
            frows_v[r, pl.ds(48, 16)] = plsc.bitcast(v1 & hi, jnp.float32)

        pltpu.sync_copy(frows_v, acc_sh.at[dst_v.at[i]], add=True)

    fire(0, rows_a, gsem_a)
    plsc.subcore_barrier()

    def body(g, carry):
        i = g * 2
        fire(i + 1, rows_b, gsem_b)
        drain(i, rows_a, gsem_a, 0)

        @pl.when(i + 2 < CPW)
        def _():
            fire(i + 2, rows_a, gsem_a)

        drain(i + 1, rows_b, gsem_b, 1)
        return carry

    lax.fori_loop(0, CPW // 2, body, 0)
    plsc.subcore_barrier()
    pltpu.sync_copy(acc_sh.at[pl.ds(s * RPT, RPT)],
                    acc_out.at[c, pl.ds(s * RPT, RPT)])
    pltpu.sync_copy(deg_sh.at[pl.ds(s * RPT, RPT)],
                    deg_out.at[c, pl.ds(s * RPT, RPT)])


def _sc_aggregate(x0, x1, src3d, dst3d, zacc, zdeg, ones):
    mesh = plsc.VectorSubcoreMesh(core_axis_name="c", subcore_axis_name="s")
    out_type = (jax.ShapeDtypeStruct((NC, NPAD, DH), jnp.float32),
                jax.ShapeDtypeStruct((NC, NPAD, 16), jnp.float32))
    kern = pl.kernel(
        _sc_body,
        out_type=out_type,
        mesh=mesh,
        compiler_params=pltpu.CompilerParams(use_tc_tiling_on_sc=False,
                                            needs_layout_passes=False),
        scratch_types=[
            pltpu.VMEM((CPW, CH), jnp.int32),
            pltpu.VMEM((CPW, CH), jnp.int32),
            pltpu.VMEM((CH, DH), jnp.bfloat16),
            pltpu.VMEM((CH, DH), jnp.bfloat16),
            pltpu.VMEM((CH, DH), jnp.float32),
            pltpu.VMEM((CH, 16), jnp.float32),
            pltpu.VMEM_SHARED((NPAD, DH), jnp.float32),
            pltpu.VMEM_SHARED((NPAD, 16), jnp.float32),
            pltpu.SemaphoreType.DMA,
            pltpu.SemaphoreType.DMA,
        ],
    )
    return kern(x0, x1, src3d, dst3d, zacc, zdeg, ones)


def _tc_self_body(x_ref, ws_ref, b_ref, o_ref):
    o_ref[...] = jnp.dot(x_ref[...], ws_ref[...],
                         preferred_element_type=jnp.float32,
                         precision=lax.Precision.HIGHEST) + b_ref[...]


def _tc_self(x, W_self, b2d):
    blk = 1000
    return pl.pallas_call(
        _tc_self_body,
        grid=(N // blk,),
        in_specs=[
            pl.BlockSpec((blk, D), lambda i: (i, 0)),
            pl.BlockSpec((D, D), lambda i: (0, 0)),
            pl.BlockSpec((1, D), lambda i: (0, 0)),
        ],
        out_specs=pl.BlockSpec((blk, D), lambda i: (i, 0)),
        out_shape=jax.ShapeDtypeStruct((N, D), jnp.float32),
    )(x, W_self, b2d)


def _tc_body(y_ref, acc_ref, deg_ref, wn_ref, o_ref):
    deg = deg_ref[0, :, 0:1] + deg_ref[1, :, 0:1]
    inv = 1.0 / jnp.maximum(deg, 1.0)
    mean = jnp.concatenate([acc_ref[0], acc_ref[1]], axis=1) * inv
    o_ref[...] = y_ref[...] + jnp.dot(mean, wn_ref[...],
                                      preferred_element_type=jnp.float32,
                                      precision=lax.Precision.HIGHEST)


def _tc_combine(y, acc, deg, W_neigh):
    blk = 1000
    grid = (N // blk,)
    return pl.pallas_call(
        _tc_body,
        grid=grid,
        in_specs=[
            pl.BlockSpec((blk, D), lambda i: (i, 0)),
            pl.BlockSpec((NC, blk, DH), lambda i: (0, i, 0)),
            pl.BlockSpec((NC, blk, 16), lambda i: (0, i, 0)),
            pl.BlockSpec((D, D), lambda i: (0, 0)),
        ],
        out_specs=pl.BlockSpec((blk, D), lambda i: (i, 0)),
        out_shape=jax.ShapeDtypeStruct((N, D), jnp.float32),
    )(y, acc, deg, W_neigh)


def kernel(x, edge_index, W_self, W_neigh, b):
    ei = edge_index.astype(jnp.int32)
    pad = EPAD - E
    ei = jnp.concatenate(
        [ei, jnp.concatenate([jnp.zeros((1, pad), jnp.int32),
                              jnp.full((1, pad), N, jnp.int32)])], axis=1)
    src3d = ei[0].reshape(NS, CPW, CH)
    dst3d = ei[1].reshape(NS, CPW, CH)
    x0 = x[:, :DH].astype(jnp.bfloat16)
    x1 = x[:, DH:].astype(jnp.bfloat16)
    half = jnp.concatenate([jnp.arange(0, 32, 2), jnp.arange(1, 32, 2),
                            jnp.arange(32, 64, 2), jnp.arange(33, 64, 2)])
    wperm = jnp.concatenate([half, DH + half])
    W_neigh = W_neigh[wperm, :]
    zacc = jnp.zeros((RPT, DH), jnp.float32)
    zdeg = jnp.zeros((RPT, 16), jnp.float32)
    ones = jnp.ones((CH, 16), jnp.float32)
    y = _tc_self(x, W_self, b.reshape(1, D))
    acc, deg = _sc_aggregate(x0, x1, src3d, dst3d, zacc, zdeg, ones)
    return _tc_combine(y, acc, deg, W_neigh)


# speedup vs baseline: 1.8430x; 1.0036x over previous
import jax
import jax.numpy as jnp
from jax import lax
from jax.experimental import pallas as pl
from jax.experimental.pallas import tpu as pltpu
from jax.experimental.pallas import tpu_sc as plsc

N = 10000
E = 320000
D = 128
DH = D // 2
NC, NS = 2, 16
CH = 128
CPW = 158
EPAD = NS * CPW * CH
NPAD = 10112
RPT = NPAD // NS


def _sc_body(x0_hbm, x1_hbm, src_hbm, dst_hbm, zacc_hbm, zdeg_hbm, ones_hbm,
             acc_out, deg_out,
             src_v, dst_v, rows_a, rows_b, rows_c, frows_v, ones_v,
             acc_sh, deg_sh, gsem_a, gsem_b, gsem_c):
    c = lax.axis_index("c")
    s = lax.axis_index("s")
    pltpu.sync_copy(zacc_hbm, acc_sh.at[pl.ds(s * RPT, RPT)])
    pltpu.sync_copy(zdeg_hbm, deg_sh.at[pl.ds(s * RPT, RPT)])
    pltpu.sync_copy(src_hbm.at[s], src_v)
    pltpu.sync_copy(dst_hbm.at[s], dst_v)
    pltpu.sync_copy(ones_hbm, ones_v)

    def fire(i, rows, sem):
        @pl.when(c == 0)
        def _():
            pltpu.async_copy(x0_hbm.at[src_v.at[i]], rows, sem)

        @pl.when(c == 1)
        def _():
            pltpu.async_copy(x1_hbm.at[src_v.at[i]], rows, sem)

    def drain(i, rows, sem, par):
        @pl.when(par == c)
        def _():
            pltpu.sync_copy(ones_v, deg_sh.at[dst_v.at[i]], add=True)

        pltpu.make_async_copy(x0_hbm.at[src_v.at[i]], rows, sem).wait()

        @plsc.parallel_loop(0, CH, unroll=8)
        def _(r):
            v0 = plsc.bitcast(rows[r, pl.ds(0, 32)], jnp.int32)
            v1 = plsc.bitcast(rows[r, pl.ds(32, 32)], jnp.int32)
            hi = jnp.int32(-65536)
            frows_v[r, pl.ds(0, 16)] = plsc.bitcast(v0 << 16, jnp.float32)
            frows_v[r, pl.ds(16, 16)] = plsc.bitcast(v0 & hi, jnp.float32)
            frows_v[r, pl.ds(32, 16)] = plsc.bitcast(v1 << 16, jnp.float32)
            frows_v[r, pl.ds(48, 16)] = plsc.bitcast(v1 & hi, jnp.float32)

        pltpu.sync_copy(frows_v, acc_sh.at[dst_v.at[i]], add=True)

    fire(0, rows_a, gsem_a)
    fire(1, rows_b, gsem_b)
    plsc.subcore_barrier()

    def body(g, carry):
        i = g * 3
        fire(i + 2, rows_c, gsem_c)
        drain(i, rows_a, gsem_a, lax.rem(i, 2))
        fire(i + 3, rows_a, gsem_a)
        drain(i + 1, rows_b, gsem_b, lax.rem(i + 1, 2))
        fire(i + 4, rows_b, gsem_b)
        drain(i + 2, rows_c, gsem_c, lax.rem(i, 2))
        return carry

    lax.fori_loop(0, CPW // 3, body, 0)
    i = (CPW // 3) * 3
    drain(i, rows_a, gsem_a, lax.rem(i, 2))
    drain(i + 1, rows_b, gsem_b, lax.rem(i + 1, 2))
    plsc.subcore_barrier()
    pltpu.sync_copy(acc_sh.at[pl.ds(s * RPT, RPT)],
                    acc_out.at[c, pl.ds(s * RPT, RPT)])
    pltpu.sync_copy(deg_sh.at[pl.ds(s * RPT, RPT)],
                    deg_out.at[c, pl.ds(s * RPT, RPT)])


def _sc_aggregate(x0, x1, src3d, dst3d, zacc, zdeg, ones):
    mesh = plsc.VectorSubcoreMesh(core_axis_name="c", subcore_axis_name="s")
    out_type = (jax.ShapeDtypeStruct((NC, NPAD, DH), jnp.float32),
                jax.ShapeDtypeStruct((NC, NPAD, 16), jnp.float32))
    kern = pl.kernel(
        _sc_body,
        out_type=out_type,
        mesh=mesh,
        compiler_params=pltpu.CompilerParams(use_tc_tiling_on_sc=False,
                                            needs_layout_passes=False),
        scratch_types=[
            pltpu.VMEM((CPW, CH), jnp.int32),
            pltpu.VMEM((CPW, CH), jnp.int32),
            pltpu.VMEM((CH, DH), jnp.bfloat16),
            pltpu.VMEM((CH, DH), jnp.bfloat16),
            pltpu.VMEM((CH, DH), jnp.bfloat16),
            pltpu.VMEM((CH, DH), jnp.float32),
            pltpu.VMEM((CH, 16), jnp.float32),
            pltpu.VMEM_SHARED((NPAD, DH), jnp.float32),
            pltpu.VMEM_SHARED((NPAD, 16), jnp.float32),
            pltpu.SemaphoreType.DMA,
            pltpu.SemaphoreType.DMA,
            pltpu.SemaphoreType.DMA,
        ],
    )
    return kern(x0, x1, src3d, dst3d, zacc, zdeg, ones)


def _tc_self_body(x_ref, ws_ref, b_ref, o_ref):
    o_ref[...] = jnp.dot(x_ref[...], ws_ref[...],
                         preferred_element_type=jnp.float32,
                         precision=lax.Precision.HIGHEST) + b_ref[...]


def _tc_self(x, W_self, b2d):
    blk = 1000
    return pl.pallas_call(
        _tc_self_body,
        grid=(N // blk,),
        in_specs=[
            pl.BlockSpec((blk, D), lambda i: (i, 0)),
            pl.BlockSpec((D, D), lambda i: (0, 0)),
            pl.BlockSpec((1, D), lambda i: (0, 0)),
        ],
        out_specs=pl.BlockSpec((blk, D), lambda i: (i, 0)),
        out_shape=jax.ShapeDtypeStruct((N, D), jnp.float32),
    )(x, W_self, b2d)


def _tc_body(y_ref, acc_ref, deg_ref, wn_ref, o_ref):
    deg = deg_ref[0, :, 0:1] + deg_ref[1, :, 0:1]
    inv = 1.0 / jnp.maximum(deg, 1.0)
    mean = jnp.concatenate([acc_ref[0], acc_ref[1]], axis=1) * inv
    o_ref[...] = y_ref[...] + jnp.dot(mean, wn_ref[...],
                                      preferred_element_type=jnp.float32,
                                      precision=lax.Precision.HIGHEST)


def _tc_combine(y, acc, deg, W_neigh):
    blk = 1000
    grid = (N // blk,)
    return pl.pallas_call(
        _tc_body,
        grid=grid,
        in_specs=[
            pl.BlockSpec((blk, D), lambda i: (i, 0)),
            pl.BlockSpec((NC, blk, DH), lambda i: (0, i, 0)),
            pl.BlockSpec((NC, blk, 16), lambda i: (0, i, 0)),
            pl.BlockSpec((D, D), lambda i: (0, 0)),
        ],
        out_specs=pl.BlockSpec((blk, D), lambda i: (i, 0)),
        out_shape=jax.ShapeDtypeStruct((N, D), jnp.float32),
    )(y, acc, deg, W_neigh)


def kernel(x, edge_index, W_self, W_neigh, b):
    ei = edge_index.astype(jnp.int32)
    pad = EPAD - E
    ei = jnp.concatenate(
        [ei, jnp.concatenate([jnp.zeros((1, pad), jnp.int32),
                              jnp.full((1, pad), N, jnp.int32)])], axis=1)
    src3d = ei[0].reshape(NS, CPW, CH)
    dst3d = ei[1].reshape(NS, CPW, CH)
    x0 = x[:, :DH].astype(jnp.bfloat16)
    x1 = x[:, DH:].astype(jnp.bfloat16)
    half = jnp.concatenate([jnp.arange(0, 32, 2), jnp.arange(1, 32, 2),
                            jnp.arange(32, 64, 2), jnp.arange(33, 64, 2)])
    wperm = jnp.concatenate([half, DH + half])
    W_neigh = W_neigh[wperm, :]
    zacc = jnp.zeros((RPT, DH), jnp.float32)
    zdeg = jnp.zeros((RPT, 16), jnp.float32)
    ones = jnp.ones((CH, 16), jnp.float32)
    y = _tc_self(x, W_self, b.reshape(1, D))
    acc, deg = _sc_aggregate(x0, x1, src3d, dst3d, zacc, zdeg, ones)
    return _tc_combine(y, acc, deg, W_neigh)
